# baseline (speedup 1.0000x reference)
import jax
import jax.numpy as jnp
from jax import lax
from jax.experimental import pallas as pl
from jax.experimental.pallas import tpu as pltpu
from jax.experimental.pallas import tpu_sc as plsc

B, L, D = 4096, 200, 64
NC, NS = 2, 16
NW = NC * NS
TOK = B * L
RPW = B // NW
NCH = RPW
POSN = L * D
LGS = tuple(range(0, 192, 16)) + (184,)


def _body(emb_hbm, x_hbm, pos_hbm, out_hbm,
          pos_lin, posT,
          idx0, idx1, q0, q1, pb0, pb1, r0, r1, t0, t1,
          i0, i1, g0, g1, o0, o1):
    idx = (idx0, idx1)
    q = (q0, q1)
    pb = (pb0, pb1)
    rows = (r0, r1)
    tout = (t0, t1)
    isem = (i0, i1)
    gsem = (g0, g1)
    osem = (o0, o1)

    wid = lax.axis_index("s") * NC + lax.axis_index("c")
    row_w = wid * RPW

    ii = lax.iota(jnp.int32, 16)
    ii64 = ii * 64

    def idx_dma(c, s):
        pltpu.async_copy(x_hbm.at[pl.ds((row_w + c) * L, L)], idx[s], isem[s])

    def idx_wait(c, s):
        pltpu.make_async_copy(x_hbm.at[pl.ds((row_w + c) * L, L)], idx[s],
                              isem[s]).wait()

    def prep(s):
        for lg in LGS:
            v = idx[s][pl.ds(lg, 16)]
            q[s][pl.ds(lg, 16)] = lax.shift_right_logical(v, 1)
            pb[s][pl.ds(lg, 16)] = (v & 1) * 64

    def gather(c, s):
        pltpu.async_copy(emb_hbm.at[q[s]], rows[s], gsem[s])

    def gather_wait(c, s):
        pltpu.make_async_copy(emb_hbm.at[q[s]], rows[s], gsem[s]).wait()

    def out_dma(c, s):
        pltpu.async_copy(tout[s], out_hbm.at[row_w + c], osem[s])

    def out_wait(c, s):
        pltpu.make_async_copy(tout[s], out_hbm.at[row_w + c],
                              osem[s]).wait()

    def transpose_add(s):
        for lg in LGS:
            lrow = ii + lg
            pbc = pb[s][pl.ds(lg, 16)]

            @pl.loop(0, D, unroll=4)
            def _d(d):
                col = pbc + d
                g = plsc.load_gather(rows[s], [lrow, col])
                tout[s][d, pl.ds(lg, 16)] = g + posT[d, pl.ds(lg, 16)]

    pltpu.sync_copy(pos_hbm.at[pl.ds(0, POSN)], pos_lin)
    for lg in LGS:
        base = ii64 + lg * 64

        @pl.loop(0, D, unroll=4)
        def _pd(d):
            posT[d, pl.ds(lg, 16)] = plsc.load_gather(pos_lin, [base + d])

    def _step(c, s, o):
        @pl.when(c + 1 < NCH)
        def _launch():
            idx_wait(c + 1, o)
            prep(o)
            gather(c + 1, o)

        gather_wait(c, s)

        @pl.when(c >= 2)
        def _drain():
            out_wait(c - 2, s)

        transpose_add(s)
        out_dma(c, s)

        @pl.when(c + 2 < NCH)
        def _next_idx():
            idx_dma(c + 2, s)

    idx_dma(0, 0)
    idx_wait(0, 0)
    prep(0)
    gather(0, 0)
    idx_dma(1, 1)

    @pl.loop(0, NCH, step=2)
    def _chunk(c):
        _step(c, 0, 1)
        _step(c + 1, 1, 0)

    out_wait(NCH - 2, (NCH - 2) % 2)
    out_wait(NCH - 1, (NCH - 1) % 2)


@jax.jit
def kernel(x, embed_table, pos_table):
    emb2 = embed_table.reshape(500000, 128)
    x_flat = x.reshape(TOK).astype(jnp.int32)
    pos_flat = pos_table.reshape(-1)
    mesh = plsc.VectorSubcoreMesh(core_axis_name="c", subcore_axis_name="s",
                                  num_cores=NC, num_subcores=NS)
    out = pl.kernel(
        _body,
        out_type=jax.ShapeDtypeStruct((B, D, L), jnp.float32),
        mesh=mesh,
        compiler_params=pltpu.CompilerParams(use_tc_tiling_on_sc=True,
                                             needs_layout_passes=False),
        scratch_types=[
            pltpu.VMEM((POSN,), jnp.float32),
            pltpu.VMEM((D, L), jnp.float32),
            pltpu.VMEM((L,), jnp.int32),
            pltpu.VMEM((L,), jnp.int32),
            pltpu.VMEM((L,), jnp.int32),
            pltpu.VMEM((L,), jnp.int32),
            pltpu.VMEM((L,), jnp.int32),
            pltpu.VMEM((L,), jnp.int32),
            pltpu.VMEM((L, 128), jnp.float32),
            pltpu.VMEM((L, 128), jnp.float32),
            pltpu.VMEM((D, L), jnp.float32),
            pltpu.VMEM((D, L), jnp.float32),
            pltpu.SemaphoreType.DMA,
            pltpu.SemaphoreType.DMA,
            pltpu.SemaphoreType.DMA,
            pltpu.SemaphoreType.DMA,
            pltpu.SemaphoreType.DMA,
            pltpu.SemaphoreType.DMA,
        ],
    )(emb2, x_flat, pos_flat)
    return out.transpose(0, 2, 1)
